# speedup vs baseline: 1.2422x; 1.2422x over previous
import functools

import jax
import jax.numpy as jnp
from jax.experimental import pallas as pl

N = 10000
E = 320000
R = 2000
P = 5000
D = 128
H = 135
EM = E + N

BLK = 1024
EMP = ((EM + BLK - 1) // BLK) * BLK


def _edge_mlp_body(z1_ref, w2_ref, b2_ref, out_ref):
    h1 = jnp.tanh(z1_ref[0])
    z2 = jax.lax.dot_general(h1, w2_ref[0], (((1,), (0,)), ((), ())),
                             preferred_element_type=jnp.float32)
    out_ref[0] = jnp.tanh(z2 + b2_ref[0])


def _edge_mlp(z1, W2s, b2s):
    grid = (3, EMP // BLK)
    return pl.pallas_call(
        _edge_mlp_body,
        grid=grid,
        in_specs=[
            pl.BlockSpec((1, BLK, H), lambda c, i: (c, i, 0)),
            pl.BlockSpec((1, H, 128), lambda c, i: (c, 0, 0)),
            pl.BlockSpec((1, 1, 128), lambda c, i: (c, 0, 0)),
        ],
        out_specs=pl.BlockSpec((1, BLK, 128), lambda c, i: (c, i, 0)),
        out_shape=jax.ShapeDtypeStruct((3, EMP, 128), jnp.float32),
    )(z1, W2s, b2s)


def _sincos_angle(v1, v2):
    c = jnp.cross(v1, v2)
    cn2 = jnp.sum(c * c, axis=1)
    d = jnp.sum(v1 * v2, axis=1)
    bad = (cn2 == 0) & (jnp.abs(d) < 1e-12)
    d_safe = jnp.where(bad, 1.0, d)
    h = jnp.sqrt(cn2 + d_safe * d_safe)
    cn = jnp.sqrt(cn2)
    return cn / h, d_safe / h


def kernel(x, pos, normal, edge_index2, edge_index3, edge_index4,
           atom_to_residue, src_idx, tgt_idx,
           W21, b21, W22, b22, W31, b31, W32, b32, W41, b41, W42, b42,
           Wa, ba, Wr, br, Wl, bl):
    loops = jnp.arange(N, dtype=edge_index2.dtype)
    radii = (5.0, 8.5, 10.0)
    eis = (edge_index2, edge_index3, edge_index4)
    W1s = (W21, W31, W41)
    b1s = (b21, b31, b41)

    z1_all = []
    dst_all = []
    for c in range(3):
        ei = eis[c]
        src = jnp.concatenate([ei[0], loops])
        dst = jnp.concatenate([ei[1], loops])
        W1 = W1s[c]
        xw = x @ W1[:D] + b1s[c]
        pos_i = pos[dst]
        pos_j = pos[src]
        n_i = normal[dst]
        n_j = normal[src]
        pseudo = pos_j - pos_i
        s1, c1 = _sincos_angle(n_i, pseudo)
        s2, c2 = _sincos_angle(n_j, pseudo)
        s3, c3 = _sincos_angle(n_i, n_j)
        pn = jnp.sqrt(jnp.sum(pseudo * pseudo, axis=1))
        feat = jnp.stack([pn / radii[c], s1, c1, s2, c2, s3, c3], axis=1)
        z1 = xw[src] + feat @ W1[D:]
        z1 = jnp.pad(z1, ((0, EMP - EM), (0, 0)))
        z1_all.append(z1)
        dst_all.append(dst)

    z1s = jnp.stack(z1_all)
    W2s = jnp.stack((W22, W32, W42))
    b2s = jnp.stack((b22, b32, b42)).reshape(3, 1, 128)
    msg = _edge_mlp(z1s, W2s, b2s)

    outs = []
    for c in range(3):
        m = msg[c, :EM]
        outs.append(jax.ops.segment_max(m, dst_all[c], num_segments=N))
    atom_x = jnp.tanh(jnp.concatenate(outs, axis=1) @ Wa + ba)
    res_x = jax.ops.segment_max(atom_x, atom_to_residue, num_segments=R)
    res_x = jnp.where(jnp.isfinite(res_x), res_x, 0.0)
    res_x = jnp.tanh(res_x @ Wr + br)
    x_s = res_x[src_idx]
    x_t = res_x[tgt_idx]
    return (x_s - x_t) @ Wl + bl


# speedup vs baseline: 1.9700x; 1.5858x over previous
import functools

import jax
import jax.numpy as jnp
from jax import lax
from jax.experimental import pallas as pl
from jax.experimental.pallas import tpu as pltpu, tpu_sc as plsc

N = 10000
E = 320000
R = 2000
P = 5000
D = 128
HP = 144

BLK = 1024
EM = E + N
EMP = ((EM + BLK - 1) // BLK) * BLK
TOT = 3 * EMP

NW = 32
CH = 512
SUP = 1024
NSUP = TOT // SUP
NRND = -(-NSUP // NW)

_MESH = plsc.VectorSubcoreMesh(core_axis_name="c", subcore_axis_name="s",
                               num_cores=2, num_subcores=16)


def _invsqrt(x, approx_sqrt):
    y = 1.0 / jnp.maximum(approx_sqrt, 1e-20)
    for _ in range(3):
        y = y * (1.5 - 0.5 * x * y * y)
    return y


def _norm3(ax, ay, az):
    x = ax * ax + ay * ay + az * az
    aax, aay, aaz = jnp.abs(ax), jnp.abs(ay), jnp.abs(az)
    m = jnp.maximum(jnp.maximum(aax, aay), aaz)
    s = aax + aay + aaz
    return x * _invsqrt(x, 0.92 * m + 0.34 * (s - m))


def _sincos(ux, uy, uz, vx, vy, vz):
    cx = uy * vz - uz * vy
    cy = uz * vx - ux * vz
    cz = ux * vy - uy * vx
    cn = _norm3(cx, cy, cz)
    cn2 = cx * cx + cy * cy + cz * cz
    d = ux * vx + uy * vy + uz * vz
    bad = (cn2 == 0.0) & (jnp.abs(d) < 1e-12)
    dsafe = jnp.where(bad, 1.0, d)
    h2 = cn2 + dsafe * dsafe
    ad = jnp.abs(dsafe)
    m = jnp.maximum(cn, ad)
    inv_h = _invsqrt(h2, 0.96 * m + 0.4 * (cn + ad - m))
    return cn * inv_h, dsafe * inv_h


def _sc_gather_body(xw_hbm, px_h, py_h, pz_h, nx_h, ny_h, nz_h,
                    srcn_hbm, didx_hbm,
                    xwg_out, f0_o, f1_o, f2_o, f3_o, f4_o, f5_o, f6_o,
                    idx_a, idx_b, xw_buf,
                    o0, o1, o2, o3, o4, o5, o6,
                    fsx, fsy, fsz, fnsx, fnsy, fnsz,
                    fdx, fdy, fdz, fndx, fndy, fndz, sem, semx):
    wid = lax.axis_index("s") * 2 + lax.axis_index("c")
    sbufs = (fsx, fsy, fsz, fnsx, fnsy, fnsz)
    dbufs = (fdx, fdy, fdz, fndx, fndy, fndz)
    tbls = (px_h, py_h, pz_h, nx_h, ny_h, nz_h)
    obufs = (o0, o1, o2, o3, o4, o5, o6)
    fouts = (f0_o, f1_o, f2_o, f3_o, f4_o, f5_o, f6_o)

    def feat_grp(g, _):
        o = pl.ds(g * 16, 16)
        psx, psy, psz = fsx[o], fsy[o], fsz[o]
        nsx, nsy, nsz = fnsx[o], fnsy[o], fnsz[o]
        pdx, pdy, pdz = fdx[o], fdy[o], fdz[o]
        ndx, ndy, ndz = fndx[o], fndy[o], fndz[o]
        px, py, pz = psx - pdx, psy - pdy, psz - pdz
        s1, c1 = _sincos(ndx, ndy, ndz, px, py, pz)
        s2, c2 = _sincos(nsx, nsy, nsz, px, py, pz)
        s3, c3 = _sincos(ndx, ndy, ndz, nsx, nsy, nsz)
        pn = _norm3(px, py, pz)
        for buf, v in zip(obufs, (pn, s1, c1, s2, c2, s3, c3)):
            buf[o] = v
        return 0

    def chunk(t, _):
        j = jnp.minimum(wid + t * NW, NSUP - 1)
        r0 = j * (SUP // 128)
        pltpu.sync_copy(srcn_hbm.at[pl.ds(r0, 8)], idx_a)
        pltpu.sync_copy(didx_hbm.at[pl.ds(r0, 8)], idx_b)
        gh = []
        for kk in range(8):
            o = pl.ds(kk * 128, 128)
            for f in range(6):
                gh.append(pltpu.async_copy(
                    tbls[f].at[idx_a.at[kk]], sbufs[f].at[o], sem))
                gh.append(pltpu.async_copy(
                    tbls[f].at[idx_b.at[kk]], dbufs[f].at[o], sem))
        for h in range(2):
            hs = []
            for k in range(4):
                hs.append(pltpu.async_copy(
                    xw_hbm.at[idx_a.at[4 * h + k]],
                    xw_buf.at[pl.ds(k * 128, 128)], semx))
            for hh in hs:
                hh.wait()
            pltpu.sync_copy(xw_buf,
                            xwg_out.at[pl.ds(j * SUP + h * CH, CH)])
        for hh in gh:
            hh.wait()
        lax.fori_loop(0, SUP // 16, feat_grp, 0)
        for f in range(7):
            pltpu.sync_copy(obufs[f], fouts[f].at[pl.ds(j * SUP, SUP)])
        return 0

    lax.fori_loop(0, NRND, chunk, 0)


@functools.partial(
    pl.kernel,
    out_type=[jax.ShapeDtypeStruct((TOT, 128), jnp.float32)]
    + [jax.ShapeDtypeStruct((TOT,), jnp.float32)] * 7,
    mesh=_MESH,
    scratch_types=[
        pltpu.VMEM((8, 128), jnp.int32),
        pltpu.VMEM((8, 128), jnp.int32),
        pltpu.VMEM((CH, 128), jnp.float32),
    ] + [pltpu.VMEM((SUP,), jnp.float32)] * 19 + [
        pltpu.SemaphoreType.DMA,
        pltpu.SemaphoreType.DMA,
    ],
)
def _sc_gather(*refs):
    _sc_gather_body(*refs)


def _edge_mlp_body(xg_ref, fp_ref, w1_ref, w2_ref, b2_ref, out_ref):
    cat = jnp.concatenate([xg_ref[0], fp_ref[0]], axis=1)
    z1 = jax.lax.dot_general(cat, w1_ref[0], (((1,), (0,)), ((), ())),
                             preferred_element_type=jnp.float32)
    h1 = jnp.tanh(z1)
    z2 = jax.lax.dot_general(h1, w2_ref[0], (((1,), (0,)), ((), ())),
                             preferred_element_type=jnp.float32)
    out_ref[0] = jnp.tanh(z2 + b2_ref[0])


def _edge_mlp(xg, fp, W1s, W2s, b2s):
    grid = (3, EMP // BLK)
    return pl.pallas_call(
        _edge_mlp_body,
        grid=grid,
        in_specs=[
            pl.BlockSpec((1, BLK, 128), lambda c, i: (c, i, 0)),
            pl.BlockSpec((1, BLK, 8), lambda c, i: (c, i, 0)),
            pl.BlockSpec((1, 136, HP), lambda c, i: (c, 0, 0)),
            pl.BlockSpec((1, HP, 128), lambda c, i: (c, 0, 0)),
            pl.BlockSpec((1, 1, 128), lambda c, i: (c, 0, 0)),
        ],
        out_specs=pl.BlockSpec((1, BLK, 128), lambda c, i: (c, i, 0)),
        out_shape=jax.ShapeDtypeStruct((3, EMP, 128), jnp.float32),
    )(xg, fp, W1s, W2s, b2s)


def kernel(x, pos, normal, edge_index2, edge_index3, edge_index4,
           atom_to_residue, src_idx, tgt_idx,
           W21, b21, W22, b22, W31, b31, W32, b32, W41, b41, W42, b42,
           Wa, ba, Wr, br, Wl, bl):
    loops = jnp.arange(N, dtype=jnp.int32)
    radii = (5.0, 8.5, 10.0)
    eis = (edge_index2, edge_index3, edge_index4)
    W1s = (W21, W31, W41)
    b1s = (b21, b31, b41)
    W2s_in = (W22, W32, W42)
    b2s_in = (b22, b32, b42)

    w1_list, w2_list, src_list, dst_list = [], [], [], []
    for c in range(3):
        W1 = W1s[c]
        w1 = jnp.concatenate([W1, b1s[c][None]], axis=0)
        w1_list.append(jnp.pad(w1, ((0, 0), (0, HP - 135))))
        w2_list.append(jnp.pad(W2s_in[c], ((0, HP - 135), (0, 0))))
        ei = eis[c].astype(jnp.int32)
        src = jnp.concatenate([ei[0], loops])
        dst = jnp.concatenate([ei[1], loops])
        src_list.append(jnp.pad(src, (0, EMP - EM)))
        dst_list.append(jnp.pad(dst, (0, EMP - EM)))

    srcs = jnp.stack(src_list)
    dsts = jnp.stack(dst_list)
    srcn = srcs.reshape(TOT // 128, 128)
    didx = dsts.reshape(TOT // 128, 128)

    res = _sc_gather(x, pos[:, 0], pos[:, 1], pos[:, 2],
                     normal[:, 0], normal[:, 1], normal[:, 2],
                     srcn, didx)
    xg = res[0].reshape(3, EMP, 128)
    rad = jnp.array(radii, jnp.float32).repeat(EMP)
    fp = jnp.stack([res[1] / rad] + list(res[2:])
                   + [jnp.ones((TOT,), jnp.float32)],
                   axis=1).reshape(3, EMP, 8)

    msg = _edge_mlp(xg, fp, jnp.stack(w1_list),
                    jnp.stack(w2_list), jnp.stack(b2s_in).reshape(3, 1, 128))

    outs = []
    for c in range(3):
        outs.append(jax.ops.segment_max(msg[c, :EM], dst_list[c][:EM],
                                        num_segments=N))
    atom_x = jnp.tanh(jnp.concatenate(outs, axis=1) @ Wa + ba)
    res_x = jax.ops.segment_max(atom_x, atom_to_residue, num_segments=R)
    res_x = jnp.where(jnp.isfinite(res_x), res_x, 0.0)
    res_x = jnp.tanh(res_x @ Wr + br)
    x_s = res_x[src_idx]
    x_t = res_x[tgt_idx]
    return (x_s - x_t) @ Wl + bl


# speedup vs baseline: 2.3153x; 1.1753x over previous
import functools

import jax
import jax.numpy as jnp
from jax import lax
from jax.experimental import pallas as pl
from jax.experimental.pallas import tpu as pltpu, tpu_sc as plsc

N = 10000
E = 320000
R = 2000
P = 5000
D = 128
HP = 144

BLK = 1024
EM = E + N
EMP = ((EM + BLK - 1) // BLK) * BLK
TOT = 3 * EMP

NW = 32
CH = 512
SUP = 1024
NSUP = TOT // SUP
NRND = -(-NSUP // NW)

_MESH = plsc.VectorSubcoreMesh(core_axis_name="c", subcore_axis_name="s",
                               num_cores=2, num_subcores=16)


def _invsqrt(x, approx_sqrt):
    y = 1.0 / jnp.maximum(approx_sqrt, 1e-20)
    for _ in range(3):
        y = y * (1.5 - 0.5 * x * y * y)
    return y


def _norm3(ax, ay, az):
    x = ax * ax + ay * ay + az * az
    aax, aay, aaz = jnp.abs(ax), jnp.abs(ay), jnp.abs(az)
    m = jnp.maximum(jnp.maximum(aax, aay), aaz)
    s = aax + aay + aaz
    return x * _invsqrt(x, 0.92 * m + 0.34 * (s - m))


def _sincos(ux, uy, uz, vx, vy, vz):
    cx = uy * vz - uz * vy
    cy = uz * vx - ux * vz
    cz = ux * vy - uy * vx
    cn = _norm3(cx, cy, cz)
    cn2 = cx * cx + cy * cy + cz * cz
    d = ux * vx + uy * vy + uz * vz
    bad = (cn2 == 0.0) & (jnp.abs(d) < 1e-12)
    dsafe = jnp.where(bad, 1.0, d)
    h2 = cn2 + dsafe * dsafe
    ad = jnp.abs(dsafe)
    m = jnp.maximum(cn, ad)
    inv_h = _invsqrt(h2, 0.96 * m + 0.4 * (cn + ad - m))
    return cn * inv_h, dsafe * inv_h


def _sc_gather_body(xw_hbm, px_h, py_h, pz_h, nx_h, ny_h, nz_h,
                    srcn_hbm, didx_hbm,
                    xwg_out, f0_o, f1_o, f2_o, f3_o, f4_o, f5_o, f6_o,
                    idx_a, idx_b, xw_buf,
                    o0, o1, o2, o3, o4, o5, o6,
                    fsx, fsy, fsz, fnsx, fnsy, fnsz,
                    fdx, fdy, fdz, fndx, fndy, fndz, sem, semx):
    wid = lax.axis_index("s") * 2 + lax.axis_index("c")
    sbufs = (fsx, fsy, fsz, fnsx, fnsy, fnsz)
    dbufs = (fdx, fdy, fdz, fndx, fndy, fndz)
    tbls = (px_h, py_h, pz_h, nx_h, ny_h, nz_h)
    obufs = (o0, o1, o2, o3, o4, o5, o6)
    fouts = (f0_o, f1_o, f2_o, f3_o, f4_o, f5_o, f6_o)

    def feat_grp(g, _):
        o = pl.ds(g * 16, 16)
        psx, psy, psz = fsx[o], fsy[o], fsz[o]
        nsx, nsy, nsz = fnsx[o], fnsy[o], fnsz[o]
        pdx, pdy, pdz = fdx[o], fdy[o], fdz[o]
        ndx, ndy, ndz = fndx[o], fndy[o], fndz[o]
        px, py, pz = psx - pdx, psy - pdy, psz - pdz
        s1, c1 = _sincos(ndx, ndy, ndz, px, py, pz)
        s2, c2 = _sincos(nsx, nsy, nsz, px, py, pz)
        s3, c3 = _sincos(ndx, ndy, ndz, nsx, nsy, nsz)
        pn = _norm3(px, py, pz)
        for buf, v in zip(obufs, (pn, s1, c1, s2, c2, s3, c3)):
            buf[o] = v
        return 0

    def chunk(t, _):
        j = jnp.minimum(wid + t * NW, NSUP - 1)
        r0 = j * (SUP // 128)
        pltpu.sync_copy(srcn_hbm.at[pl.ds(r0, 8)], idx_a)
        pltpu.sync_copy(didx_hbm.at[pl.ds(r0, 8)], idx_b)
        gh = []
        for kk in range(8):
            o = pl.ds(kk * 128, 128)
            for f in range(6):
                gh.append(pltpu.async_copy(
                    tbls[f].at[idx_a.at[kk]], sbufs[f].at[o], sem))
                gh.append(pltpu.async_copy(
                    tbls[f].at[idx_b.at[kk]], dbufs[f].at[o], sem))
        for h in range(2):
            hs = []
            for k in range(4):
                hs.append(pltpu.async_copy(
                    xw_hbm.at[idx_a.at[4 * h + k]],
                    xw_buf.at[pl.ds(k * 128, 128)], semx))
            for hh in hs:
                hh.wait()
            pltpu.sync_copy(xw_buf,
                            xwg_out.at[pl.ds(j * SUP + h * CH, CH)])
        for hh in gh:
            hh.wait()
        lax.fori_loop(0, SUP // 16, feat_grp, 0)
        for f in range(7):
            pltpu.sync_copy(obufs[f], fouts[f].at[pl.ds(j * SUP, SUP)])
        return 0

    lax.fori_loop(0, NRND, chunk, 0)


@functools.partial(
    pl.kernel,
    out_type=[jax.ShapeDtypeStruct((TOT, 128), jnp.float32)]
    + [jax.ShapeDtypeStruct((TOT,), jnp.float32)] * 7,
    mesh=_MESH,
    scratch_types=[
        pltpu.VMEM((8, 128), jnp.int32),
        pltpu.VMEM((8, 128), jnp.int32),
        pltpu.VMEM((CH, 128), jnp.float32),
    ] + [pltpu.VMEM((SUP,), jnp.float32)] * 19 + [
        pltpu.SemaphoreType.DMA,
        pltpu.SemaphoreType.DMA,
    ],
)
def _sc_gather(*refs):
    _sc_gather_body(*refs)


def _edge_mlp_body(xg_ref, fp_ref, w1_ref, w2_ref, b2_ref, out_ref):
    cat = jnp.concatenate([xg_ref[0], fp_ref[0]], axis=1)
    z1 = jax.lax.dot_general(cat, w1_ref[0], (((1,), (0,)), ((), ())),
                             preferred_element_type=jnp.float32)
    h1 = jnp.tanh(z1)
    z2 = jax.lax.dot_general(h1, w2_ref[0], (((1,), (0,)), ((), ())),
                             preferred_element_type=jnp.float32)
    out_ref[0] = jnp.tanh(z2 + b2_ref[0])


def _edge_mlp(xg, fp, W1s, W2s, b2s):
    grid = (3, EMP // BLK)
    return pl.pallas_call(
        _edge_mlp_body,
        grid=grid,
        in_specs=[
            pl.BlockSpec((1, BLK, 128), lambda c, i: (c, i, 0)),
            pl.BlockSpec((1, BLK, 8), lambda c, i: (c, i, 0)),
            pl.BlockSpec((1, 136, HP), lambda c, i: (c, 0, 0)),
            pl.BlockSpec((1, HP, 128), lambda c, i: (c, 0, 0)),
            pl.BlockSpec((1, 1, 128), lambda c, i: (c, 0, 0)),
        ],
        out_specs=pl.BlockSpec((1, BLK, 128), lambda c, i: (c, i, 0)),
        out_shape=jax.ShapeDtypeStruct((3, EMP, 128), jnp.float32),
    )(xg, fp, W1s, W2s, b2s)


def kernel(x, pos, normal, edge_index2, edge_index3, edge_index4,
           atom_to_residue, src_idx, tgt_idx,
           W21, b21, W22, b22, W31, b31, W32, b32, W41, b41, W42, b42,
           Wa, ba, Wr, br, Wl, bl):
    loops = jnp.arange(N, dtype=jnp.int32)
    radii = (5.0, 8.5, 10.0)
    eis = (edge_index2, edge_index3, edge_index4)
    W1s = (W21, W31, W41)
    b1s = (b21, b31, b41)
    W2s_in = (W22, W32, W42)
    b2s_in = (b22, b32, b42)

    w1_list, w2_list, src_list, dst_list = [], [], [], []
    for c in range(3):
        W1 = W1s[c]
        w1 = jnp.concatenate([W1, b1s[c][None]], axis=0)
        w1_list.append(jnp.pad(w1, ((0, 0), (0, HP - 135))))
        w2_list.append(jnp.pad(W2s_in[c], ((0, HP - 135), (0, 0))))
        ei = eis[c].astype(jnp.int32)
        src = jnp.concatenate([ei[0], loops])
        dst = jnp.concatenate([ei[1], loops])
        src_list.append(jnp.pad(src, (0, EMP - EM)))
        dst_list.append(jnp.pad(dst, (0, EMP - EM)))

    srcs = jnp.stack(src_list)
    dsts = jnp.stack(dst_list)
    srcn = srcs.reshape(TOT // 128, 128)
    didx = dsts.reshape(TOT // 128, 128)

    res = _sc_gather(x, pos[:, 0], pos[:, 1], pos[:, 2],
                     normal[:, 0], normal[:, 1], normal[:, 2],
                     srcn, didx)
    xg = res[0].reshape(3, EMP, 128)
    rad = jnp.array(radii, jnp.float32).repeat(EMP)
    fp = jnp.stack([res[1] / rad] + list(res[2:])
                   + [jnp.ones((TOT,), jnp.float32)],
                   axis=1).reshape(3, EMP, 8)

    msg = _edge_mlp(xg, fp, jnp.stack(w1_list),
                    jnp.stack(w2_list), jnp.stack(b2s_in).reshape(3, 1, 128))

    outs = []
    for c in range(3):
        dseg = dst_list[c].at[EM:].set(N)
        outs.append(jax.ops.segment_max(msg[c], dseg, num_segments=N))
    atom_x = jnp.tanh(jnp.concatenate(outs, axis=1) @ Wa + ba)
    res_x = jax.ops.segment_max(atom_x, atom_to_residue, num_segments=R)
    res_x = jnp.where(jnp.isfinite(res_x), res_x, 0.0)
    res_x = jnp.tanh(res_x @ Wr + br)
    x_s = res_x[src_idx]
    x_t = res_x[tgt_idx]
    return (x_s - x_t) @ Wl + bl
